# baseline (speedup 1.0000x reference)
import jax
import jax.numpy as jnp
from jax import lax
from jax.experimental import pallas as pl
from jax.experimental.pallas import tpu as pltpu
from jax.experimental.pallas import tpu_sc as plsc

BATCH = 16384
EMBED_DIM = 16
BUCKETS = 100000
NUM_CORES = 2
NUM_SUBCORES = 16
NUM_WORKERS = NUM_CORES * NUM_SUBCORES
B_PER_W = BATCH // NUM_WORKERS
LANES = 16
NWORDS = B_PER_W * EMBED_DIM
NCHUNK = 4
C_ROWS = B_PER_W // NCHUNK
C_WORDS = C_ROWS * EMBED_DIM


def _fm_body(uid_hbm, tid_hbm, utab_hbm, itab_hbm, wb_hbm, out_hbm,
             idx_u_v, idx_t_v, fid_u_v, fid_t_v, gu_v, gt_v, out_v,
             wb_v, su0, su1, su2, su3, st0, st1, st2, st3):
    su = (su0, su1, su2, su3)
    st = (st0, st1, st2, st3)
    wid = lax.axis_index("s") * NUM_CORES + lax.axis_index("c")
    base = wid * B_PER_W

    pltpu.sync_copy(uid_hbm.at[pl.ds(base, B_PER_W)], idx_u_v)
    pltpu.sync_copy(tid_hbm.at[pl.ds(base, B_PER_W)], idx_t_v)
    pltpu.sync_copy(wb_hbm, wb_v)

    def build_fid(c):
        def body(j, carry):
            iu = idx_u_v[pl.ds(c * C_ROWS + j * LANES, LANES)]
            it = idx_t_v[pl.ds(c * C_ROWS + j * LANES, LANES)]
            for d in range(EMBED_DIM):
                s = pl.ds(c * C_WORDS + d * C_ROWS + j * LANES, LANES)
                fid_u_v[s] = iu + (d * BUCKETS)
                fid_t_v[s] = it + (d * BUCKETS)
            return carry
        lax.fori_loop(0, C_ROWS // LANES, body, 0)

    copies = []
    for c in range(NCHUNK):
        build_fid(c)
        sl = pl.ds(c * C_WORDS, C_WORDS)
        cu = pltpu.async_copy(utab_hbm.at[fid_u_v.at[sl]], gu_v.at[sl], su[c])
        ct = pltpu.async_copy(itab_hbm.at[fid_t_v.at[sl]], gt_v.at[sl], st[c])
        copies.append((cu, ct))

    wv = wb_v[pl.ds(0, LANES)]
    bv = wb_v[pl.ds(LANES, LANES)]

    for c in range(NCHUNK):
        cu, ct = copies[c]
        cu.wait()
        ct.wait()

        def grp(j, carry):
            acc = jnp.zeros((LANES,), jnp.float32)
            for d in range(EMBED_DIM):
                s = pl.ds(c * C_WORDS + d * C_ROWS + j * LANES, LANES)
                acc = acc + gu_v[s] * gt_v[s]
            z = acc * wv + bv
            y = 1.0 / (1.0 + jnp.exp(-z))
            out_v[pl.ds(c * C_ROWS + j * LANES, LANES)] = y
            return carry

        lax.fori_loop(0, C_ROWS // LANES, grp, 0)

    pltpu.sync_copy(out_v, out_hbm.at[pl.ds(base, B_PER_W)])


@jax.jit
def _fm_sc(f_uid, f_tid, utab_flat, itab_flat, wb):
    mesh = plsc.VectorSubcoreMesh(core_axis_name="c", subcore_axis_name="s")
    return pl.kernel(
        _fm_body,
        out_type=jax.ShapeDtypeStruct((BATCH,), jnp.float32),
        mesh=mesh,
        compiler_params=pltpu.CompilerParams(needs_layout_passes=False),
        scratch_types=[
            pltpu.VMEM((B_PER_W,), jnp.int32),
            pltpu.VMEM((B_PER_W,), jnp.int32),
            pltpu.VMEM((NWORDS,), jnp.int32),
            pltpu.VMEM((NWORDS,), jnp.int32),
            pltpu.VMEM((NWORDS,), jnp.float32),
            pltpu.VMEM((NWORDS,), jnp.float32),
            pltpu.VMEM((B_PER_W,), jnp.float32),
            pltpu.VMEM((8 * LANES,), jnp.float32),
            pltpu.SemaphoreType.DMA,
            pltpu.SemaphoreType.DMA,
            pltpu.SemaphoreType.DMA,
            pltpu.SemaphoreType.DMA,
            pltpu.SemaphoreType.DMA,
            pltpu.SemaphoreType.DMA,
            pltpu.SemaphoreType.DMA,
            pltpu.SemaphoreType.DMA,
        ],
    )(f_uid, f_tid, utab_flat, itab_flat, wb)


def kernel(f_uid, f_tid, user_table, item_table, W, b):
    utab_flat = user_table.T.reshape(-1)
    itab_flat = item_table.T.reshape(-1)
    wb = jnp.concatenate([
        jnp.broadcast_to(W.reshape(1), (LANES,)),
        jnp.broadcast_to(b.reshape(1), (LANES,)),
        jnp.zeros((8 * LANES - 2 * LANES,), jnp.float32),
    ])
    y = _fm_sc(f_uid, f_tid, utab_flat, itab_flat, wb)
    return y.reshape(BATCH, 1)
